# baseline (speedup 1.0000x reference)
import math

import jax
import jax.numpy as jnp
from jax.experimental import pallas as pl
from jax.experimental.pallas import tpu as pltpu

D_MODEL = 1024
N_HEAD = 16
DT = D_MODEL // N_HEAD
SEQ = 2048
TOPU = 40
ROW_TILE = 512
COL_CHUNK = 512


def _qproj_kernel(x_ref, w_ref, b_ref, o_ref):
    x = x_ref[...]
    w = w_ref[...]
    acc = jax.lax.dot_general(x, w, (((1,), (1,)), ((), ())),
                              preferred_element_type=jnp.float32)
    o_ref[...] = acc + b_ref[0][None, :]


def _kvproj_kernel(k_ref, v_ref, wk_ref, wv_ref, bk_ref, bv_ref,
                   ko_ref, vo_ref):
    xk = k_ref[0]
    xv = v_ref[0]
    wk = wk_ref[...]
    wv = wv_ref[...]
    kk = jax.lax.dot_general(xk, wk, (((1,), (1,)), ((), ())),
                             preferred_element_type=jnp.float32)
    vv = jax.lax.dot_general(xv, wv, (((1,), (1,)), ((), ())),
                             preferred_element_type=jnp.float32)
    ko_ref[0] = kk + bk_ref[0, 0][None, :]
    vo_ref[0] = vv + bv_ref[0, 0][None, :]


def _attn_kernel(q_ref, k_ref, v_ref, mask_ref, counts_ref, o_ref, qr_ref):
    Q = q_ref[0]
    m = jnp.full((SEQ, 1), -jnp.inf, jnp.float32)
    acc = jnp.zeros((SEQ, 1), jnp.float32)
    for c in range(SEQ // COL_CHUNK):
        Kc = k_ref[0, pl.ds(c * COL_CHUNK, COL_CHUNK), :]
        Sc = jax.lax.dot_general(Q, Kc, (((1,), (1,)), ((), ())),
                                 preferred_element_type=jnp.float32)
        mask_c = mask_ref[0, pl.ds(c * COL_CHUNK, COL_CHUNK)]
        cnt_c = counts_ref[0, pl.ds(c * COL_CHUNK, COL_CHUNK)]
        m = jnp.maximum(m, jnp.max(Sc + mask_c[None, :], axis=1,
                                   keepdims=True))
        acc = acc + jnp.sum(Sc * cnt_c[None, :], axis=1, keepdims=True)
    M = m - acc * (1.0 / SEQ)

    iota = jax.lax.broadcasted_iota(jnp.int32, (SEQ, 1), 0)

    def step(j, Mc):
        vmin = jnp.min(Mc)
        idx = jnp.min(jnp.where(Mc == vmin, iota, jnp.int32(SEQ)))
        qr_ref[pl.ds(j, 1), :] = q_ref[0, pl.ds(idx, 1), :]
        return jnp.where(iota == idx, jnp.inf, Mc)

    jax.lax.fori_loop(0, TOPU, step, M, unroll=False)

    Qr = qr_ref[...]
    K = k_ref[0]
    scores = jax.lax.dot_general(Qr, K, (((1,), (1,)), ((), ())),
                                 preferred_element_type=jnp.float32)
    scores = scores * (1.0 / math.sqrt(DT))
    smax = jnp.max(scores, axis=1, keepdims=True)
    p = jnp.exp(scores - smax)
    p = p / jnp.sum(p, axis=1, keepdims=True)
    o_ref[0] = jnp.dot(p, v_ref[0], preferred_element_type=jnp.float32)


def _oproj_kernel(x_ref, w_ref, b_ref, o_ref):
    acc = jax.lax.dot_general(x_ref[...], w_ref[...], (((1,), (1,)), ((), ())),
                              preferred_element_type=jnp.float32)
    o_ref[...] = acc + b_ref[0][None, :]


def kernel(q, k, v, Wq, bq, Wk, bk, Wv, bv, Wc, bc):
    B, L, D = q.shape
    H, dt = N_HEAD, DT
    BH = B * H

    index_sample = jax.random.randint(jax.random.key(1234), (SEQ,), 0, SEQ)
    counts = jnp.zeros((SEQ,), jnp.float32).at[index_sample].add(1.0)
    maskbias = jnp.where(counts > 0, 0.0, -jnp.inf).astype(jnp.float32)
    counts2 = counts.reshape(1, SEQ)
    mask2 = maskbias.reshape(1, SEQ)

    x_q = q.reshape(B * L, D)
    qp = pl.pallas_call(
        _qproj_kernel,
        grid=(B * L // ROW_TILE,),
        in_specs=[
            pl.BlockSpec((ROW_TILE, D), lambda i: (i, 0)),
            pl.BlockSpec((D, D), lambda i: (0, 0)),
            pl.BlockSpec((1, D), lambda i: (0, 0)),
        ],
        out_specs=pl.BlockSpec((ROW_TILE, D), lambda i: (i, 0)),
        out_shape=jax.ShapeDtypeStruct((B * L, D), jnp.float32),
    )(x_q, Wq, bq.reshape(1, D))
    Qh = qp.reshape(BH, L, dt)

    LT = L // ROW_TILE
    Kh, Vh = pl.pallas_call(
        _kvproj_kernel,
        grid=(B, LT, H),
        in_specs=[
            pl.BlockSpec((1, ROW_TILE, D), lambda b, t, h: (b, t, 0)),
            pl.BlockSpec((1, ROW_TILE, D), lambda b, t, h: (b, t, 0)),
            pl.BlockSpec((DT, D), lambda b, t, h: (h, 0)),
            pl.BlockSpec((DT, D), lambda b, t, h: (h, 0)),
            pl.BlockSpec((1, 1, DT), lambda b, t, h: (h, 0, 0)),
            pl.BlockSpec((1, 1, DT), lambda b, t, h: (h, 0, 0)),
        ],
        out_specs=[
            pl.BlockSpec((1, ROW_TILE, dt), lambda b, t, h: (b * H + h, t, 0)),
            pl.BlockSpec((1, ROW_TILE, dt), lambda b, t, h: (b * H + h, t, 0)),
        ],
        out_shape=[
            jax.ShapeDtypeStruct((BH, L, dt), jnp.float32),
            jax.ShapeDtypeStruct((BH, L, dt), jnp.float32),
        ],
    )(k, v, Wk, Wv, bk.reshape(H, 1, dt), bv.reshape(H, 1, dt))

    out_h = pl.pallas_call(
        _attn_kernel,
        grid=(BH,),
        in_specs=[
            pl.BlockSpec((1, L, dt), lambda i: (i, 0, 0)),
            pl.BlockSpec((1, L, dt), lambda i: (i, 0, 0)),
            pl.BlockSpec((1, L, dt), lambda i: (i, 0, 0)),
            pl.BlockSpec((1, SEQ), lambda i: (0, 0)),
            pl.BlockSpec((1, SEQ), lambda i: (0, 0)),
        ],
        out_specs=pl.BlockSpec((1, TOPU, dt), lambda i: (i, 0, 0)),
        out_shape=jax.ShapeDtypeStruct((BH, TOPU, dt), jnp.float32),
        scratch_shapes=[pltpu.VMEM((TOPU, dt), jnp.float32)],
    )(Qh, Kh, Vh, mask2, counts2)

    o = out_h.reshape(B, H, TOPU, dt)
    o = jnp.swapaxes(o, 1, 2).reshape(B * TOPU, H * dt)
    out = pl.pallas_call(
        _oproj_kernel,
        grid=(1,),
        in_specs=[
            pl.BlockSpec((B * TOPU, D), lambda i: (0, 0)),
            pl.BlockSpec((D, D), lambda i: (0, 0)),
            pl.BlockSpec((1, D), lambda i: (0, 0)),
        ],
        out_specs=pl.BlockSpec((B * TOPU, D), lambda i: (0, 0)),
        out_shape=jax.ShapeDtypeStruct((B * TOPU, D), jnp.float32),
    )(o, Wc, bc.reshape(1, D))
    return out.reshape(B, TOPU, D)


# speedup vs baseline: 3.6021x; 3.6021x over previous
import math

import numpy as np
import jax
import jax.numpy as jnp
from jax.experimental import pallas as pl
from jax.experimental.pallas import tpu as pltpu

D_MODEL = 1024
N_HEAD = 16
DT = D_MODEL // N_HEAD
SEQ = 2048
TOPU = 40
ROW_TILE = 512
COL_CHUNK = 512

_IDX_SAMPLE = np.asarray(
    jax.random.randint(jax.random.key(1234), (SEQ,), 0, SEQ))
_COUNTS = np.zeros((SEQ,), np.float32)
np.add.at(_COUNTS, _IDX_SAMPLE, 1.0)
_MASKBIAS = np.where(_COUNTS > 0, 0.0, -np.inf).astype(np.float32)


def _qproj_kernel(x_ref, w_ref, b_ref, o_ref):
    acc = jax.lax.dot_general(x_ref[...], w_ref[...], (((1,), (1,)), ((), ())),
                              preferred_element_type=jnp.float32)
    o_ref[...] = acc + b_ref[0][None, :]


def _kvproj_kernel(k_ref, v_ref, wk_ref, wv_ref, bk_ref, bv_ref,
                   ko_ref, vo_ref):
    kk = jax.lax.dot_general(k_ref[0], wk_ref[...], (((1,), (1,)), ((), ())),
                             preferred_element_type=jnp.float32)
    vv = jax.lax.dot_general(v_ref[0], wv_ref[...], (((1,), (1,)), ((), ())),
                             preferred_element_type=jnp.float32)
    ko_ref[0] = kk + bk_ref[0, 0][None, :]
    vo_ref[0] = vv + bv_ref[0, 0][None, :]


def _stats_kernel(q_ref, k_ref, mask_ref, cnt_ref, m_ref):
    Q = q_ref[0]
    m = jnp.full((1, SEQ), -jnp.inf, jnp.float32)
    acc = jnp.zeros((1, SEQ), jnp.float32)
    for c in range(SEQ // COL_CHUNK):
        Kc = k_ref[0, pl.ds(c * COL_CHUNK, COL_CHUNK), :]
        St = jax.lax.dot_general(Kc, Q, (((1,), (1,)), ((), ())),
                                 preferred_element_type=jnp.float32)
        mask_c = mask_ref[pl.ds(c * COL_CHUNK, COL_CHUNK), :]
        cnt_c = cnt_ref[pl.ds(c * COL_CHUNK, COL_CHUNK), :]
        m = jnp.maximum(m, jnp.max(St + mask_c, axis=0, keepdims=True))
        acc = acc + jnp.sum(St * cnt_c, axis=0, keepdims=True)
    m_ref[0] = m - acc * (1.0 / SEQ)


def _select_kernel(m_ref, oh_ref):
    M = m_ref[:, 0, :]
    col = jax.lax.broadcasted_iota(jnp.int32, M.shape, 1)
    for j in range(TOPU):
        vmin = jnp.min(M, axis=1, keepdims=True)
        idx = jnp.min(jnp.where(M == vmin, col, jnp.int32(SEQ)),
                      axis=1, keepdims=True)
        hit = col == idx
        oh_ref[:, j, :] = hit.astype(jnp.float32)
        M = jnp.where(hit, jnp.inf, M)


def _attn_kernel(oh_ref, q_ref, k_ref, v_ref, o_ref):
    Qr = jnp.dot(oh_ref[0], q_ref[0], preferred_element_type=jnp.float32)
    scores = jax.lax.dot_general(Qr, k_ref[0], (((1,), (1,)), ((), ())),
                                 preferred_element_type=jnp.float32)
    scores = scores * (1.0 / math.sqrt(DT))
    smax = jnp.max(scores, axis=1, keepdims=True)
    p = jnp.exp(scores - smax)
    p = p / jnp.sum(p, axis=1, keepdims=True)
    o_ref[0] = jnp.dot(p, v_ref[0], preferred_element_type=jnp.float32)


def _oproj_kernel(x_ref, w_ref, b_ref, o_ref):
    acc = jax.lax.dot_general(x_ref[...], w_ref[...], (((1,), (1,)), ((), ())),
                              preferred_element_type=jnp.float32)
    o_ref[...] = acc + b_ref[0][None, :]


def kernel(q, k, v, Wq, bq, Wk, bk, Wv, bv, Wc, bc):
    B, L, D = q.shape
    H, dt = N_HEAD, DT
    BH = B * H
    maskcol = jnp.asarray(_MASKBIAS).reshape(SEQ, 1)
    cntcol = jnp.asarray(_COUNTS).reshape(SEQ, 1)

    qp = pl.pallas_call(
        _qproj_kernel,
        grid=(B * L // ROW_TILE,),
        in_specs=[
            pl.BlockSpec((ROW_TILE, D), lambda i: (i, 0)),
            pl.BlockSpec((D, D), lambda i: (0, 0)),
            pl.BlockSpec((1, D), lambda i: (0, 0)),
        ],
        out_specs=pl.BlockSpec((ROW_TILE, D), lambda i: (i, 0)),
        out_shape=jax.ShapeDtypeStruct((B * L, D), jnp.float32),
        compiler_params=pltpu.CompilerParams(
            dimension_semantics=("parallel",)),
    )(q.reshape(B * L, D), Wq, bq.reshape(1, D))
    Qh = qp.reshape(BH, L, dt)

    LT = L // ROW_TILE
    Kh, Vh = pl.pallas_call(
        _kvproj_kernel,
        grid=(B, LT, H),
        in_specs=[
            pl.BlockSpec((1, ROW_TILE, D), lambda b, t, h: (b, t, 0)),
            pl.BlockSpec((1, ROW_TILE, D), lambda b, t, h: (b, t, 0)),
            pl.BlockSpec((DT, D), lambda b, t, h: (h, 0)),
            pl.BlockSpec((DT, D), lambda b, t, h: (h, 0)),
            pl.BlockSpec((1, 1, DT), lambda b, t, h: (h, 0, 0)),
            pl.BlockSpec((1, 1, DT), lambda b, t, h: (h, 0, 0)),
        ],
        out_specs=[
            pl.BlockSpec((1, ROW_TILE, dt), lambda b, t, h: (b * H + h, t, 0)),
            pl.BlockSpec((1, ROW_TILE, dt), lambda b, t, h: (b * H + h, t, 0)),
        ],
        out_shape=[
            jax.ShapeDtypeStruct((BH, L, dt), jnp.float32),
            jax.ShapeDtypeStruct((BH, L, dt), jnp.float32),
        ],
        compiler_params=pltpu.CompilerParams(
            dimension_semantics=("parallel", "parallel", "arbitrary")),
    )(k, v, Wk, Wv, bk.reshape(H, 1, dt), bv.reshape(H, 1, dt))

    M = pl.pallas_call(
        _stats_kernel,
        grid=(BH,),
        in_specs=[
            pl.BlockSpec((1, L, dt), lambda i: (i, 0, 0)),
            pl.BlockSpec((1, L, dt), lambda i: (i, 0, 0)),
            pl.BlockSpec((SEQ, 1), lambda i: (0, 0)),
            pl.BlockSpec((SEQ, 1), lambda i: (0, 0)),
        ],
        out_specs=pl.BlockSpec((1, 1, SEQ), lambda i: (i, 0, 0)),
        out_shape=jax.ShapeDtypeStruct((BH, 1, SEQ), jnp.float32),
        compiler_params=pltpu.CompilerParams(
            dimension_semantics=("parallel",)),
    )(Qh, Kh, maskcol, cntcol)

    onehot = pl.pallas_call(
        _select_kernel,
        grid=(1,),
        in_specs=[pl.BlockSpec((BH, 1, SEQ), lambda i: (0, 0, 0))],
        out_specs=pl.BlockSpec((BH, TOPU, SEQ), lambda i: (0, 0, 0)),
        out_shape=jax.ShapeDtypeStruct((BH, TOPU, SEQ), jnp.float32),
    )(M)

    out_h = pl.pallas_call(
        _attn_kernel,
        grid=(BH,),
        in_specs=[
            pl.BlockSpec((1, TOPU, SEQ), lambda i: (i, 0, 0)),
            pl.BlockSpec((1, L, dt), lambda i: (i, 0, 0)),
            pl.BlockSpec((1, L, dt), lambda i: (i, 0, 0)),
            pl.BlockSpec((1, L, dt), lambda i: (i, 0, 0)),
        ],
        out_specs=pl.BlockSpec((1, TOPU, dt), lambda i: (i, 0, 0)),
        out_shape=jax.ShapeDtypeStruct((BH, TOPU, dt), jnp.float32),
        compiler_params=pltpu.CompilerParams(
            dimension_semantics=("parallel",)),
    )(onehot, Qh, Kh, Vh)

    o = out_h.reshape(B, H, TOPU, dt)
    o = jnp.swapaxes(o, 1, 2).reshape(B * TOPU, H * dt)
    out = pl.pallas_call(
        _oproj_kernel,
        grid=(1,),
        in_specs=[
            pl.BlockSpec((B * TOPU, D), lambda i: (0, 0)),
            pl.BlockSpec((D, D), lambda i: (0, 0)),
            pl.BlockSpec((1, D), lambda i: (0, 0)),
        ],
        out_specs=pl.BlockSpec((B * TOPU, D), lambda i: (0, 0)),
        out_shape=jax.ShapeDtypeStruct((B * TOPU, D), jnp.float32),
    )(o, Wc, bc.reshape(1, D))
    return out.reshape(B, TOPU, D)


# speedup vs baseline: 5.8518x; 1.6245x over previous
import math

import numpy as np
import jax
import jax.numpy as jnp
from jax.experimental import pallas as pl
from jax.experimental.pallas import tpu as pltpu

D_MODEL = 1024
N_HEAD = 16
DT = D_MODEL // N_HEAD
SEQ = 2048
TOPU = 40
ROW_TILE = 512
COL_CHUNK = 512

_COUNTS_STR = (
    "0101223320101013011111401101120020000010202210020011210240021020"
    "0203020320100000300100100020200131220221010522102001021031122010"
    "4211302100011241111111201010001110232101002111110010211202030220"
    "0121000301231011001003210020011312003010000340330031200310100100"
    "4120221140222123113011010010502001012032002111222102222011300020"
    "1131010142110201511120110111011130130000150121421012112012410001"
    "0201011112015001001111221111410212101100022202110100101001102120"
    "1130100121101011120110010211140020311110002001102113120220221001"
    "1211320011122100001202001112300102210110101001231110110031001001"
    "0010100022201002021110201201012101122121031010103230200111010211"
    "1011000220002010231521020101010012001231002301102100131100000130"
    "3101123001012010002031132210131221510002110130201020110010111002"
    "1101110112122020111103121011110003202011021101000120011212000111"
    "1202134001011411110102102100202102100111022211312011002103211221"
    "1201011111022111110112000022131011011020002102011021010112001311"
    "2320112200110210000013041011111312201012210020020301020000120010"
    "0301201121010010002101222214040001010100230111210101011111122010"
    "0102101221200210012210112110000102200321111420030012012221201212"
    "0110122101200123210212114100122121142010000210102011123001122001"
    "0001111020200002232000020101001211212031212112111020311000101011"
    "1200001100100121011001232620220011131100232010022000021120011002"
    "2102201201203010031001110110000111300022111111211212110100003130"
    "0020202010111101011003000112101123113100100021002131221314211100"
    "1101000110010114313103310010100025101100011012013101111114212100"
    "1100201321112020000151121000101223423022010010321212111220213101"
    "1110100102001000022105310400010111011002110201100211210200130120"
    "0110042010000301210102221031010100002112100101202000000113341102"
    "0100021011102121014211113011101014113110100212111001022230011213"
    "2021100114300102002211011230010001300043250223101020102020101000"
    "0102120100112012110110101110011201113230101122200211111011000300"
    "0000030312120010001012220010011111202110211201020111001131010011"
    "2131020111112010121203102102010100103111011211001041000331002100"
)
_COUNTS = (np.frombuffer(_COUNTS_STR.encode(), np.uint8)
           - ord("0")).astype(np.float32)
_MASKBIAS = np.where(_COUNTS > 0, 0.0, -np.inf).astype(np.float32)


def _proj_kernel(q_ref, k_ref, v_ref, wq_ref, wk_ref, wv_ref,
                 bq_ref, bk_ref, bv_ref, qo_ref, ko_ref, vo_ref):
    dn = (((1,), (1,)), ((), ()))
    qo_ref[...] = jax.lax.dot_general(
        q_ref[...], wq_ref[...], dn,
        preferred_element_type=jnp.float32) + bq_ref[0][None, :]
    ko_ref[...] = jax.lax.dot_general(
        k_ref[...], wk_ref[...], dn,
        preferred_element_type=jnp.float32) + bk_ref[0][None, :]
    vo_ref[...] = jax.lax.dot_general(
        v_ref[...], wv_ref[...], dn,
        preferred_element_type=jnp.float32) + bv_ref[0][None, :]


def _stats_kernel(qe_ref, qo_ref, kp_ref, mask_ref, cnt_ref,
                  me_ref, mo_ref):
    dn = (((1,), (1,)), ((), ()))
    Qe = qe_ref[0]
    Qo = qo_ref[0]
    me = jnp.full((1, SEQ), -jnp.inf, jnp.float32)
    mo = jnp.full((1, SEQ), -jnp.inf, jnp.float32)
    ae = jnp.zeros((1, SEQ), jnp.float32)
    ao = jnp.zeros((1, SEQ), jnp.float32)
    for c in range(SEQ // COL_CHUNK):
        Kc = kp_ref[0, pl.ds(c * COL_CHUNK, COL_CHUNK), :]
        Ke = Kc[:, :DT]
        Ko = Kc[:, DT:]
        mask_c = mask_ref[pl.ds(c * COL_CHUNK, COL_CHUNK), :]
        cnt_c = cnt_ref[pl.ds(c * COL_CHUNK, COL_CHUNK), :]
        Se = jax.lax.dot_general(Ke, Qe, dn,
                                 preferred_element_type=jnp.float32)
        me = jnp.maximum(me, jnp.max(Se + mask_c, axis=0, keepdims=True))
        ae = ae + jnp.sum(Se * cnt_c, axis=0, keepdims=True)
        So = jax.lax.dot_general(Ko, Qo, dn,
                                 preferred_element_type=jnp.float32)
        mo = jnp.maximum(mo, jnp.max(So + mask_c, axis=0, keepdims=True))
        ao = ao + jnp.sum(So * cnt_c, axis=0, keepdims=True)
    me_ref[0] = me - ae * (1.0 / SEQ)
    mo_ref[0] = mo - ao * (1.0 / SEQ)


def _select_kernel(m_ref, oh_ref):
    M = m_ref[:, 0, :]
    col = jax.lax.broadcasted_iota(jnp.int32, M.shape, 1)
    for j in range(TOPU):
        vmin = jnp.min(M, axis=1, keepdims=True)
        idx = jnp.min(jnp.where(M == vmin, col, jnp.int32(SEQ)),
                      axis=1, keepdims=True)
        hit = col == idx
        oh_ref[:, j, :] = hit.astype(jnp.float32)
        M = jnp.where(hit, jnp.inf, M)


def _attn_kernel(ohe_ref, oho_ref, qe_ref, qo_ref, kp_ref, vp_ref,
                 wc_ref, bc_ref, o_ref):
    h2 = pl.program_id(1)
    dn = (((1,), (1,)), ((), ()))
    K = kp_ref[0]
    V = vp_ref[0]
    ats = []
    for oh_r, q_r, lo in ((ohe_ref, qe_ref, 0), (oho_ref, qo_ref, DT)):
        Kh = K[:, lo:lo + DT]
        Vh = V[:, lo:lo + DT]
        Qr = jnp.dot(oh_r[0], q_r[0], preferred_element_type=jnp.float32)
        scores = jax.lax.dot_general(Qr, Kh, dn,
                                     preferred_element_type=jnp.float32)
        scores = scores * (1.0 / math.sqrt(DT))
        smax = jnp.max(scores, axis=1, keepdims=True)
        p = jnp.exp(scores - smax)
        p = p / jnp.sum(p, axis=1, keepdims=True)
        ats.append(jnp.dot(p, Vh, preferred_element_type=jnp.float32))
    at_pair = jnp.concatenate(ats, axis=1)
    acc = jax.lax.dot_general(at_pair, wc_ref[...], dn,
                              preferred_element_type=jnp.float32)

    @pl.when(h2 == 0)
    def _():
        o_ref[0] = acc + bc_ref[0][None, :]

    @pl.when(h2 != 0)
    def _():
        o_ref[0] = o_ref[0] + acc


def kernel(q, k, v, Wq, bq, Wk, bk, Wv, bv, Wc, bc):
    B, L, D = q.shape
    H, dt = N_HEAD, DT
    BH = B * H
    HP = H // 2
    maskcol = jnp.asarray(_MASKBIAS).reshape(SEQ, 1)
    cntcol = jnp.asarray(_COUNTS).reshape(SEQ, 1)

    qp, kp, vp = pl.pallas_call(
        _proj_kernel,
        grid=(B * L // ROW_TILE,),
        in_specs=[
            pl.BlockSpec((ROW_TILE, D), lambda i: (i, 0)),
            pl.BlockSpec((ROW_TILE, D), lambda i: (i, 0)),
            pl.BlockSpec((ROW_TILE, D), lambda i: (i, 0)),
            pl.BlockSpec((D, D), lambda i: (0, 0)),
            pl.BlockSpec((D, D), lambda i: (0, 0)),
            pl.BlockSpec((D, D), lambda i: (0, 0)),
            pl.BlockSpec((1, D), lambda i: (0, 0)),
            pl.BlockSpec((1, D), lambda i: (0, 0)),
            pl.BlockSpec((1, D), lambda i: (0, 0)),
        ],
        out_specs=[
            pl.BlockSpec((ROW_TILE, D), lambda i: (i, 0)),
            pl.BlockSpec((ROW_TILE, D), lambda i: (i, 0)),
            pl.BlockSpec((ROW_TILE, D), lambda i: (i, 0)),
        ],
        out_shape=[jax.ShapeDtypeStruct((B * L, D), jnp.float32)] * 3,
        compiler_params=pltpu.CompilerParams(
            dimension_semantics=("parallel",)),
    )(q.reshape(B * L, D), k.reshape(B * L, D), v.reshape(B * L, D),
      Wq, Wk, Wv, bq.reshape(1, D), bk.reshape(1, D), bv.reshape(1, D))

    Qh = qp.reshape(BH, L, dt)
    kp3 = kp.reshape(B, L, D)
    vp3 = vp.reshape(B, L, D)

    Me, Mo = pl.pallas_call(
        _stats_kernel,
        grid=(B, HP),
        in_specs=[
            pl.BlockSpec((1, L, dt), lambda b, p: (b * H + 2 * p, 0, 0)),
            pl.BlockSpec((1, L, dt), lambda b, p: (b * H + 2 * p + 1, 0, 0)),
            pl.BlockSpec((1, L, 2 * dt), lambda b, p: (b, 0, p)),
            pl.BlockSpec((SEQ, 1), lambda b, p: (0, 0)),
            pl.BlockSpec((SEQ, 1), lambda b, p: (0, 0)),
        ],
        out_specs=[
            pl.BlockSpec((1, 1, SEQ), lambda b, p: (b * HP + p, 0, 0)),
            pl.BlockSpec((1, 1, SEQ), lambda b, p: (b * HP + p, 0, 0)),
        ],
        out_shape=[jax.ShapeDtypeStruct((B * HP, 1, SEQ), jnp.float32)] * 2,
        compiler_params=pltpu.CompilerParams(
            dimension_semantics=("parallel", "arbitrary")),
    )(Qh, Qh, kp3, maskcol, cntcol)

    M32 = jnp.stack([Me[:, 0, :], Mo[:, 0, :]], axis=1).reshape(BH, 1, SEQ)

    onehot = pl.pallas_call(
        _select_kernel,
        grid=(1,),
        in_specs=[pl.BlockSpec((BH, 1, SEQ), lambda i: (0, 0, 0))],
        out_specs=pl.BlockSpec((BH, TOPU, SEQ), lambda i: (0, 0, 0)),
        out_shape=jax.ShapeDtypeStruct((BH, TOPU, SEQ), jnp.float32),
    )(M32)

    out = pl.pallas_call(
        _attn_kernel,
        grid=(B, HP),
        in_specs=[
            pl.BlockSpec((1, TOPU, SEQ), lambda b, p: (b * H + 2 * p, 0, 0)),
            pl.BlockSpec((1, TOPU, SEQ), lambda b, p: (b * H + 2 * p + 1, 0, 0)),
            pl.BlockSpec((1, L, dt), lambda b, p: (b * H + 2 * p, 0, 0)),
            pl.BlockSpec((1, L, dt), lambda b, p: (b * H + 2 * p + 1, 0, 0)),
            pl.BlockSpec((1, L, 2 * dt), lambda b, p: (b, 0, p)),
            pl.BlockSpec((1, L, 2 * dt), lambda b, p: (b, 0, p)),
            pl.BlockSpec((D, 2 * dt), lambda b, p: (0, p)),
            pl.BlockSpec((1, D), lambda b, p: (0, 0)),
        ],
        out_specs=pl.BlockSpec((1, TOPU, D), lambda b, p: (b, 0, 0)),
        out_shape=jax.ShapeDtypeStruct((B, TOPU, D), jnp.float32),
        compiler_params=pltpu.CompilerParams(
            dimension_semantics=("parallel", "arbitrary")),
    )(onehot, onehot, Qh, Qh, kp3, vp3, Wc, bc.reshape(1, D))
    return out
